# baseline (device time: 13090 ns/iter reference)
import jax
import jax.numpy as jnp
from jax import lax
from jax.experimental import pallas as pl
from jax.experimental.pallas import tpu as pltpu

C = 4


def kernel(partial, gamma):
    _, m2, d = partial.shape
    m = m2 // 2
    q = m // 2
    qc = q // C

    def body(p_ref, g_ref, out_ref,
             xs_ref, xr_ref, ys_ref, yr_ref,
             xs_sems, xr_sems, ys_sems, yr_sems):
        my_x = lax.axis_index("x")
        my_y = lax.axis_index("y")
        peer_x = 1 - my_x
        peer_y = 1 - my_y

        barrier_sem = pltpu.get_barrier_semaphore()
        pl.semaphore_signal(
            barrier_sem, inc=1,
            device_id=(peer_x, my_y), device_id_type=pl.DeviceIdType.MESH,
        )
        pl.semaphore_signal(
            barrier_sem, inc=1,
            device_id=(my_x, peer_y), device_id_type=pl.DeviceIdType.MESH,
        )
        pl.semaphore_wait(barrier_sem, 2)

        my_q0 = my_x * m + my_y * q
        peer_q0 = peer_x * m + my_y * q

        x_rdmas = []
        for c in range(C):
            sl = pl.ds(c * qc, qc)
            xs_ref[sl, :] = p_ref[0, pl.ds(peer_q0 + c * qc, qc), :].astype(
                jnp.bfloat16
            )
            rdma = pltpu.make_async_remote_copy(
                src_ref=xs_ref.at[sl],
                dst_ref=xr_ref.at[sl],
                send_sem=xs_sems.at[c],
                recv_sem=xr_sems.at[c],
                device_id=(peer_x, my_y),
                device_id_type=pl.DeviceIdType.MESH,
            )
            rdma.start()
            x_rdmas.append(rdma)

        y_rdmas = []
        for c in range(C):
            sl = pl.ds(c * qc, qc)
            x_rdmas[c].wait_recv()
            y_c = (
                p_ref[0, pl.ds(my_q0 + c * qc, qc), :]
                + xr_ref[sl, :].astype(jnp.float32)
            )
            ms = jnp.mean(y_c * y_c, axis=-1, keepdims=True)
            o_c = y_c * lax.rsqrt(ms + 1e-6) * g_ref[...].reshape(1, d)
            out_ref[pl.ds(my_y * q + c * qc, qc), :] = o_c
            ys_ref[sl, :] = o_c.astype(jnp.bfloat16)
            rdma = pltpu.make_async_remote_copy(
                src_ref=ys_ref.at[sl],
                dst_ref=yr_ref.at[sl],
                send_sem=ys_sems.at[c],
                recv_sem=yr_sems.at[c],
                device_id=(my_x, peer_y),
                device_id_type=pl.DeviceIdType.MESH,
            )
            rdma.start()
            y_rdmas.append(rdma)

        for c in range(C):
            sl = pl.ds(c * qc, qc)
            y_rdmas[c].wait_recv()
            out_ref[pl.ds(peer_y * q + c * qc, qc), :] = yr_ref[sl, :].astype(
                jnp.float32
            )

        for c in range(C):
            x_rdmas[c].wait_send()
            y_rdmas[c].wait_send()

    return pl.pallas_call(
        body,
        out_shape=jax.ShapeDtypeStruct((m, d), jnp.float32),
        in_specs=[
            pl.BlockSpec(memory_space=pltpu.VMEM),
            pl.BlockSpec(memory_space=pltpu.VMEM),
        ],
        out_specs=pl.BlockSpec(memory_space=pltpu.VMEM),
        scratch_shapes=[
            pltpu.VMEM((q, d), jnp.bfloat16),
            pltpu.VMEM((q, d), jnp.bfloat16),
            pltpu.VMEM((q, d), jnp.bfloat16),
            pltpu.VMEM((q, d), jnp.bfloat16),
            pltpu.SemaphoreType.DMA((C,)),
            pltpu.SemaphoreType.DMA((C,)),
            pltpu.SemaphoreType.DMA((C,)),
            pltpu.SemaphoreType.DMA((C,)),
        ],
        compiler_params=pltpu.CompilerParams(collective_id=0),
    )(partial, gamma)


# device time: 12982 ns/iter; 1.0083x vs baseline; 1.0083x over previous
import jax
import jax.numpy as jnp
from jax import lax
from jax.experimental import pallas as pl
from jax.experimental.pallas import tpu as pltpu

C = 4


def kernel(partial, gamma):
    _, m2, d = partial.shape
    m = m2 // 2
    q = m // 2
    qc = q // C

    def body(p_hbm, g_hbm, out_hbm,
             pq_vm, mq_vm, g_vm, xs_ref, xr_ref, ov_ref,
             in_sems, out_sem, xs_sems, xr_sems, ys_sems, yr_sems):
        my_x = lax.axis_index("x")
        my_y = lax.axis_index("y")
        peer_x = 1 - my_x
        peer_y = 1 - my_y
        my_q0 = my_x * m + my_y * q
        peer_q0 = peer_x * m + my_y * q

        pq_dma = pltpu.make_async_copy(
            p_hbm.at[0, pl.ds(peer_q0, q), :], pq_vm, in_sems.at[0]
        )
        mq_dma = pltpu.make_async_copy(
            p_hbm.at[0, pl.ds(my_q0, q), :], mq_vm, in_sems.at[1]
        )
        g_dma = pltpu.make_async_copy(g_hbm, g_vm, in_sems.at[2])
        pq_dma.start()
        mq_dma.start()
        g_dma.start()

        barrier_sem = pltpu.get_barrier_semaphore()
        pl.semaphore_signal(
            barrier_sem, inc=1,
            device_id=(peer_x, my_y), device_id_type=pl.DeviceIdType.MESH,
        )
        pl.semaphore_signal(
            barrier_sem, inc=1,
            device_id=(my_x, peer_y), device_id_type=pl.DeviceIdType.MESH,
        )
        pl.semaphore_wait(barrier_sem, 2)

        pq_dma.wait()
        x_rdmas = []
        for c in range(C):
            sl = pl.ds(c * qc, qc)
            xs_ref[sl, :] = pq_vm[sl, :].astype(jnp.bfloat16)
            rdma = pltpu.make_async_remote_copy(
                src_ref=xs_ref.at[sl],
                dst_ref=xr_ref.at[sl],
                send_sem=xs_sems.at[c],
                recv_sem=xr_sems.at[c],
                device_id=(peer_x, my_y),
                device_id_type=pl.DeviceIdType.MESH,
            )
            rdma.start()
            x_rdmas.append(rdma)

        mq_dma.wait()
        g_dma.wait()
        gamma_row = g_vm[...].reshape(1, d)
        y_rdmas = []
        for c in range(C):
            sl = pl.ds(c * qc, qc)
            x_rdmas[c].wait_recv()
            y_c = mq_vm[sl, :] + xr_ref[sl, :].astype(jnp.float32)
            ms = jnp.mean(y_c * y_c, axis=-1, keepdims=True)
            o_c = y_c * lax.rsqrt(ms + 1e-6) * gamma_row
            ov_ref[sl, :] = o_c.astype(jnp.bfloat16)
            rdma = pltpu.make_async_remote_copy(
                src_ref=ov_ref.at[sl],
                dst_ref=out_hbm.at[pl.ds(my_y * q + c * qc, qc), :],
                send_sem=ys_sems.at[c],
                recv_sem=yr_sems.at[c],
                device_id=(my_x, peer_y),
                device_id_type=pl.DeviceIdType.MESH,
            )
            rdma.start()
            y_rdmas.append(rdma)

        out_dma = pltpu.make_async_copy(
            ov_ref, out_hbm.at[pl.ds(my_y * q, q), :], out_sem
        )
        out_dma.start()

        for c in range(C):
            y_rdmas[c].wait_recv()
        out_dma.wait()
        for c in range(C):
            x_rdmas[c].wait_send()
            y_rdmas[c].wait_send()

    return pl.pallas_call(
        body,
        out_shape=jax.ShapeDtypeStruct((m, d), jnp.bfloat16),
        in_specs=[
            pl.BlockSpec(memory_space=pl.ANY),
            pl.BlockSpec(memory_space=pl.ANY),
        ],
        out_specs=pl.BlockSpec(memory_space=pl.ANY),
        scratch_shapes=[
            pltpu.VMEM((q, d), jnp.float32),
            pltpu.VMEM((q, d), jnp.float32),
            pltpu.VMEM((d,), jnp.float32),
            pltpu.VMEM((q, d), jnp.bfloat16),
            pltpu.VMEM((q, d), jnp.bfloat16),
            pltpu.VMEM((q, d), jnp.bfloat16),
            pltpu.SemaphoreType.DMA((3,)),
            pltpu.SemaphoreType.DMA,
            pltpu.SemaphoreType.DMA((C,)),
            pltpu.SemaphoreType.DMA((C,)),
            pltpu.SemaphoreType.DMA((C,)),
            pltpu.SemaphoreType.DMA((C,)),
        ],
        compiler_params=pltpu.CompilerParams(collective_id=0),
    )(partial, gamma)


# device time: 11508 ns/iter; 1.1375x vs baseline; 1.1281x over previous
import jax
import jax.numpy as jnp
from jax import lax
from jax.experimental import pallas as pl
from jax.experimental.pallas import tpu as pltpu

C = 4


def kernel(partial, gamma):
    _, m2, d = partial.shape
    m = m2 // 2
    q = m // 2
    qc = q // C

    def body(p_hbm, g_hbm, out_hbm,
             pq_vm, mq_vm, g_vm, xs_ref, xr_ref, ov_ref,
             in_sems, out_sem, xs_sems, xr_sems, ys_sems, yr_sems):
        my_x = lax.axis_index("x")
        my_y = lax.axis_index("y")
        peer_x = 1 - my_x
        peer_y = 1 - my_y
        my_q0 = my_x * m + my_y * q
        peer_q0 = peer_x * m + my_y * q

        pq_dma = pltpu.make_async_copy(
            p_hbm.at[0, pl.ds(peer_q0, q), :], pq_vm, in_sems.at[0]
        )
        mq_dma = pltpu.make_async_copy(
            p_hbm.at[0, pl.ds(my_q0, q), :], mq_vm, in_sems.at[1]
        )
        g_dma = pltpu.make_async_copy(g_hbm, g_vm, in_sems.at[2])
        pq_dma.start()
        mq_dma.start()
        g_dma.start()

        barrier_sem = pltpu.get_barrier_semaphore()
        pl.semaphore_signal(
            barrier_sem, inc=1,
            device_id=(peer_x, my_y), device_id_type=pl.DeviceIdType.MESH,
        )
        pl.semaphore_signal(
            barrier_sem, inc=1,
            device_id=(my_x, peer_y), device_id_type=pl.DeviceIdType.MESH,
        )
        pl.semaphore_wait(barrier_sem, 2)

        pq_dma.wait()
        x_rdmas = []
        for c in range(C):
            sl = pl.ds(c * qc, qc)
            xs_ref[sl, :] = pq_vm[sl, :].astype(jnp.bfloat16)
            rdma = pltpu.make_async_remote_copy(
                src_ref=xs_ref.at[sl],
                dst_ref=xr_ref.at[sl],
                send_sem=xs_sems.at[c],
                recv_sem=xr_sems.at[c],
                device_id=(peer_x, my_y),
                device_id_type=pl.DeviceIdType.MESH,
            )
            rdma.start()
            x_rdmas.append(rdma)

        mq_dma.wait()
        g_dma.wait()
        gamma_row = g_vm[...].reshape(1, d)
        y_rdmas = []
        for c in range(C):
            sl = pl.ds(c * qc, qc)
            x_rdmas[c].wait_recv()
            y_c = mq_vm[sl, :] + xr_ref[sl, :].astype(jnp.float32)
            ms = jnp.mean(y_c * y_c, axis=-1, keepdims=True)
            o_c = y_c * lax.rsqrt(ms + 1e-6) * gamma_row
            ov_ref[sl, :] = o_c.astype(jnp.bfloat16)
            rdma = pltpu.make_async_remote_copy(
                src_ref=ov_ref.at[sl],
                dst_ref=out_hbm.at[pl.ds(my_y * q + c * qc, qc), :],
                send_sem=ys_sems.at[c],
                recv_sem=yr_sems.at[c],
                device_id=(my_x, peer_y),
                device_id_type=pl.DeviceIdType.MESH,
            )
            rdma.start()
            y_rdmas.append(rdma)

        out_dma = pltpu.make_async_copy(
            ov_ref, out_hbm.at[pl.ds(my_y * q, q), :], out_sem
        )
        out_dma.start()

        for c in range(C):
            y_rdmas[c].wait_recv()
        out_dma.wait()
        for c in range(C):
            x_rdmas[c].wait_send()
            y_rdmas[c].wait_send()

    return pl.pallas_call(
        body,
        out_shape=jax.ShapeDtypeStruct((m, d), jnp.bfloat16),
        in_specs=[
            pl.BlockSpec(memory_space=pltpu.MemorySpace.HBM),
            pl.BlockSpec(memory_space=pltpu.MemorySpace.HBM),
        ],
        out_specs=pl.BlockSpec(memory_space=pltpu.MemorySpace.HBM),
        scratch_shapes=[
            pltpu.VMEM((q, d), jnp.float32),
            pltpu.VMEM((q, d), jnp.float32),
            pltpu.VMEM((d,), jnp.float32),
            pltpu.VMEM((q, d), jnp.bfloat16),
            pltpu.VMEM((q, d), jnp.bfloat16),
            pltpu.VMEM((q, d), jnp.bfloat16),
            pltpu.SemaphoreType.DMA((3,)),
            pltpu.SemaphoreType.DMA,
            pltpu.SemaphoreType.DMA((C,)),
            pltpu.SemaphoreType.DMA((C,)),
            pltpu.SemaphoreType.DMA((C,)),
            pltpu.SemaphoreType.DMA((C,)),
        ],
        compiler_params=pltpu.CompilerParams(collective_id=0),
    )(
        pltpu.with_memory_space_constraint(partial, pltpu.MemorySpace.HBM),
        pltpu.with_memory_space_constraint(gamma, pltpu.MemorySpace.HBM),
    )


# device time: 11379 ns/iter; 1.1504x vs baseline; 1.0113x over previous
import jax
import jax.numpy as jnp
from jax import lax
from jax.experimental import pallas as pl
from jax.experimental.pallas import tpu as pltpu

C = 4


def kernel(partial, gamma):
    _, m2, d = partial.shape
    m = m2 // 2
    q = m // 2
    qc = q // C

    def body(p_hbm, g_hbm, out_hbm,
             pq_vm, mq_vm, g_vm, xs_ref, xr_ref, ov_ref,
             pq_sems, in_sems, out_sem, xs_sems, xr_sems, ys_sems, yr_sems):
        my_x = lax.axis_index("x")
        my_y = lax.axis_index("y")
        peer_x = 1 - my_x
        peer_y = 1 - my_y
        my_q0 = my_x * m + my_y * q
        peer_q0 = peer_x * m + my_y * q

        pq_dmas = []
        for c in range(C):
            sl = pl.ds(c * qc, qc)
            dma = pltpu.make_async_copy(
                p_hbm.at[0, pl.ds(peer_q0 + c * qc, qc), :],
                pq_vm.at[sl],
                pq_sems.at[c],
            )
            dma.start()
            pq_dmas.append(dma)
        mq_dma = pltpu.make_async_copy(
            p_hbm.at[0, pl.ds(my_q0, q), :], mq_vm, in_sems.at[0]
        )
        g_dma = pltpu.make_async_copy(g_hbm, g_vm, in_sems.at[1])
        mq_dma.start()
        g_dma.start()

        barrier_sem = pltpu.get_barrier_semaphore()
        pl.semaphore_signal(
            barrier_sem, inc=1,
            device_id=(peer_x, my_y), device_id_type=pl.DeviceIdType.MESH,
        )
        pl.semaphore_signal(
            barrier_sem, inc=1,
            device_id=(my_x, peer_y), device_id_type=pl.DeviceIdType.MESH,
        )
        pl.semaphore_wait(barrier_sem, 2)

        x_rdmas = []
        for c in range(C):
            sl = pl.ds(c * qc, qc)
            pq_dmas[c].wait()
            xs_ref[sl, :] = pq_vm[sl, :].astype(jnp.bfloat16)
            rdma = pltpu.make_async_remote_copy(
                src_ref=xs_ref.at[sl],
                dst_ref=xr_ref.at[sl],
                send_sem=xs_sems.at[c],
                recv_sem=xr_sems.at[c],
                device_id=(peer_x, my_y),
                device_id_type=pl.DeviceIdType.MESH,
            )
            rdma.start()
            x_rdmas.append(rdma)

        mq_dma.wait()
        g_dma.wait()
        gamma_row = g_vm[...].reshape(1, d)
        y_rdmas = []
        for c in range(C):
            sl = pl.ds(c * qc, qc)
            x_rdmas[c].wait_recv()
            y_c = mq_vm[sl, :] + xr_ref[sl, :].astype(jnp.float32)
            ms = jnp.mean(y_c * y_c, axis=-1, keepdims=True)
            o_c = y_c * lax.rsqrt(ms + 1e-6) * gamma_row
            ov_ref[sl, :] = o_c.astype(jnp.bfloat16)
            rdma = pltpu.make_async_remote_copy(
                src_ref=ov_ref.at[sl],
                dst_ref=out_hbm.at[pl.ds(my_y * q + c * qc, qc), :],
                send_sem=ys_sems.at[c],
                recv_sem=yr_sems.at[c],
                device_id=(my_x, peer_y),
                device_id_type=pl.DeviceIdType.MESH,
            )
            rdma.start()
            y_rdmas.append(rdma)

        out_dma = pltpu.make_async_copy(
            ov_ref, out_hbm.at[pl.ds(my_y * q, q), :], out_sem
        )
        out_dma.start()

        for c in range(C):
            y_rdmas[c].wait_recv()
        out_dma.wait()
        for c in range(C):
            x_rdmas[c].wait_send()
            y_rdmas[c].wait_send()

    return pl.pallas_call(
        body,
        out_shape=jax.ShapeDtypeStruct((m, d), jnp.bfloat16),
        in_specs=[
            pl.BlockSpec(memory_space=pltpu.MemorySpace.HBM),
            pl.BlockSpec(memory_space=pltpu.MemorySpace.HBM),
        ],
        out_specs=pl.BlockSpec(memory_space=pltpu.MemorySpace.HBM),
        scratch_shapes=[
            pltpu.VMEM((q, d), jnp.float32),
            pltpu.VMEM((q, d), jnp.float32),
            pltpu.VMEM((d,), jnp.float32),
            pltpu.VMEM((q, d), jnp.bfloat16),
            pltpu.VMEM((q, d), jnp.bfloat16),
            pltpu.VMEM((q, d), jnp.bfloat16),
            pltpu.SemaphoreType.DMA((C,)),
            pltpu.SemaphoreType.DMA((2,)),
            pltpu.SemaphoreType.DMA,
            pltpu.SemaphoreType.DMA((C,)),
            pltpu.SemaphoreType.DMA((C,)),
            pltpu.SemaphoreType.DMA((C,)),
            pltpu.SemaphoreType.DMA((C,)),
        ],
        compiler_params=pltpu.CompilerParams(collective_id=0),
    )(
        pltpu.with_memory_space_constraint(partial, pltpu.MemorySpace.HBM),
        pltpu.with_memory_space_constraint(gamma, pltpu.MemorySpace.HBM),
    )


# device time: 11153 ns/iter; 1.1737x vs baseline; 1.0203x over previous
import jax
import jax.numpy as jnp
from jax import lax
from jax.experimental import pallas as pl
from jax.experimental.pallas import tpu as pltpu

C = 8


def kernel(partial, gamma):
    _, m2, d = partial.shape
    m = m2 // 2
    q = m // 2
    qc = q // C

    def body(p_hbm, g_hbm, out_hbm,
             pq_vm, mq_vm, g_vm, xs_ref, xr_ref, ov_ref,
             pq_sems, in_sems, out_sem, xs_sems, xr_sems, ys_sems, yr_sems):
        my_x = lax.axis_index("x")
        my_y = lax.axis_index("y")
        peer_x = 1 - my_x
        peer_y = 1 - my_y
        my_q0 = my_x * m + my_y * q
        peer_q0 = peer_x * m + my_y * q

        pq_dmas = []
        for c in range(C):
            sl = pl.ds(c * qc, qc)
            dma = pltpu.make_async_copy(
                p_hbm.at[0, pl.ds(peer_q0 + c * qc, qc), :],
                pq_vm.at[sl],
                pq_sems.at[c],
            )
            dma.start()
            pq_dmas.append(dma)
        mq_dma = pltpu.make_async_copy(
            p_hbm.at[0, pl.ds(my_q0, q), :], mq_vm, in_sems.at[0]
        )
        g_dma = pltpu.make_async_copy(g_hbm, g_vm, in_sems.at[1])
        mq_dma.start()
        g_dma.start()

        barrier_sem = pltpu.get_barrier_semaphore()
        pl.semaphore_signal(
            barrier_sem, inc=1,
            device_id=(peer_x, my_y), device_id_type=pl.DeviceIdType.MESH,
        )
        pl.semaphore_signal(
            barrier_sem, inc=1,
            device_id=(my_x, peer_y), device_id_type=pl.DeviceIdType.MESH,
        )
        pl.semaphore_wait(barrier_sem, 2)

        x_rdmas = []
        for c in range(C):
            sl = pl.ds(c * qc, qc)
            pq_dmas[c].wait()
            xs_ref[sl, :] = pq_vm[sl, :].astype(jnp.bfloat16)
            rdma = pltpu.make_async_remote_copy(
                src_ref=xs_ref.at[sl],
                dst_ref=xr_ref.at[sl],
                send_sem=xs_sems.at[c],
                recv_sem=xr_sems.at[c],
                device_id=(peer_x, my_y),
                device_id_type=pl.DeviceIdType.MESH,
            )
            rdma.start()
            x_rdmas.append(rdma)

        mq_dma.wait()
        g_dma.wait()
        gamma_row = g_vm[...].reshape(1, d)
        y_rdmas = []
        for c in range(C):
            sl = pl.ds(c * qc, qc)
            x_rdmas[c].wait_recv()
            y_c = mq_vm[sl, :] + xr_ref[sl, :].astype(jnp.float32)
            ms = jnp.mean(y_c * y_c, axis=-1, keepdims=True)
            o_c = y_c * lax.rsqrt(ms + 1e-6) * gamma_row
            ov_ref[sl, :] = o_c.astype(jnp.bfloat16)
            rdma = pltpu.make_async_remote_copy(
                src_ref=ov_ref.at[sl],
                dst_ref=out_hbm.at[pl.ds(my_y * q + c * qc, qc), :],
                send_sem=ys_sems.at[c],
                recv_sem=yr_sems.at[c],
                device_id=(my_x, peer_y),
                device_id_type=pl.DeviceIdType.MESH,
            )
            rdma.start()
            y_rdmas.append(rdma)

        out_dma = pltpu.make_async_copy(
            ov_ref, out_hbm.at[pl.ds(my_y * q, q), :], out_sem
        )
        out_dma.start()

        for c in range(C):
            y_rdmas[c].wait_recv()
        out_dma.wait()
        for c in range(C):
            x_rdmas[c].wait_send()
            y_rdmas[c].wait_send()

    return pl.pallas_call(
        body,
        out_shape=jax.ShapeDtypeStruct((m, d), jnp.bfloat16),
        in_specs=[
            pl.BlockSpec(memory_space=pltpu.MemorySpace.HBM),
            pl.BlockSpec(memory_space=pltpu.MemorySpace.HBM),
        ],
        out_specs=pl.BlockSpec(memory_space=pltpu.MemorySpace.HBM),
        scratch_shapes=[
            pltpu.VMEM((q, d), jnp.float32),
            pltpu.VMEM((q, d), jnp.float32),
            pltpu.VMEM((d,), jnp.float32),
            pltpu.VMEM((q, d), jnp.bfloat16),
            pltpu.VMEM((q, d), jnp.bfloat16),
            pltpu.VMEM((q, d), jnp.bfloat16),
            pltpu.SemaphoreType.DMA((C,)),
            pltpu.SemaphoreType.DMA((2,)),
            pltpu.SemaphoreType.DMA,
            pltpu.SemaphoreType.DMA((C,)),
            pltpu.SemaphoreType.DMA((C,)),
            pltpu.SemaphoreType.DMA((C,)),
            pltpu.SemaphoreType.DMA((C,)),
        ],
        compiler_params=pltpu.CompilerParams(collective_id=0),
    )(
        pltpu.with_memory_space_constraint(partial, pltpu.MemorySpace.HBM),
        pltpu.with_memory_space_constraint(gamma, pltpu.MemorySpace.HBM),
    )
